# baseline (speedup 1.0000x reference)
import jax
import jax.numpy as jnp
from jax.experimental import pallas as pl
from jax.experimental.pallas import tpu as pltpu

B, T, D = 8, 2048, 512
N = B * T
KT, KE, KD = 1024, 512, 1024
KC = KT + KE + KD + KD
TB = 1024
GRID = N // TB
COMMIT = 0.25

_F32 = jnp.float32


def _vq_argmin(f, cb, cbsq):
    rowsq = jnp.sum(f * f, axis=1, keepdims=True)
    s = jax.lax.dot_general(f, cb, (((1,), (1,)), ((), ())),
                            preferred_element_type=_F32)
    d = rowsq + cbsq - 2.0 * s
    dmin = jnp.min(d, axis=1, keepdims=True)
    iota = jax.lax.broadcasted_iota(jnp.int32, d.shape, 1)
    idx = jnp.min(jnp.where(d == dmin, iota, jnp.int32(d.shape[1])), axis=1)
    oh = (iota == idx[:, None]).astype(_F32)
    return idx, oh, jnp.sum(dmin)


def _body(x_ref, tw_ref, tb_ref, ew_ref, eb_ref, dw_ref, db_ref,
          ow_ref, ob_ref, cbt_ref, cbe_ref, cbd0_ref, cbd1_ref,
          out_ref, it_ref, ie_ref, i0_ref, i1_ref, loss_ref,
          m_ref, sq_ref):
    @pl.when(pl.program_id(0) == 0)
    def _precompute():
        ow = ow_ref[...]
        cbt = cbt_ref[...]
        cbe = cbe_ref[...]
        cbd0 = cbd0_ref[...]
        cbd1 = cbd1_ref[...]
        m_ref[0:KT] = jnp.dot(cbt, ow[0:D], preferred_element_type=_F32)
        m_ref[KT:KT + KE] = jnp.dot(cbe, ow[D:2 * D],
                                    preferred_element_type=_F32)
        m_ref[KT + KE:KT + KE + KD] = jnp.dot(
            cbd0, ow[2 * D:3 * D], preferred_element_type=_F32)
        m_ref[KT + KE + KD:KC] = jnp.dot(
            cbd1, ow[2 * D:3 * D], preferred_element_type=_F32)
        sq_ref[0, 0:KT] = jnp.sum(cbt * cbt, axis=1)
        sq_ref[0, KT:KT + KE] = jnp.sum(cbe * cbe, axis=1)
        sq_ref[0, KT + KE:KT + KE + KD] = jnp.sum(cbd0 * cbd0, axis=1)
        sq_ref[0, KT + KE + KD:KC] = jnp.sum(cbd1 * cbd1, axis=1)

    xb = x_ref[...]
    tf = jnp.dot(xb, tw_ref[...], preferred_element_type=_F32) + tb_ref[...]
    ef = jnp.dot(xb, ew_ref[...], preferred_element_type=_F32) + eb_ref[...]
    df = jnp.dot(xb, dw_ref[...], preferred_element_type=_F32) + db_ref[...]

    idx_t, oh_t, l_t = _vq_argmin(tf, cbt_ref[...], sq_ref[0:1, 0:KT])
    idx_e, oh_e, l_e = _vq_argmin(ef, cbe_ref[...], sq_ref[0:1, KT:KT + KE])

    idx0, oh0, l_0 = _vq_argmin(df, cbd0_ref[...],
                                sq_ref[0:1, KT + KE:KT + KE + KD])
    q0 = jax.lax.dot_general(oh0, cbd0_ref[...], (((1,), (0,)), ((), ())),
                             preferred_element_type=_F32,
                             precision=jax.lax.Precision.HIGHEST)
    q0st = df + (q0 - df)
    r = df - q0st
    idx1, oh1, l_1 = _vq_argmin(r, cbd1_ref[...],
                                sq_ref[0:1, KT + KE + KD:KC])

    oh_cat = jnp.concatenate([oh_t, oh_e, oh0, oh1], axis=1)
    out = jax.lax.dot_general(oh_cat, m_ref[...], (((1,), (0,)), ((), ())),
                              preferred_element_type=_F32) + ob_ref[...]
    out_ref[...] = out

    it_ref[...] = idx_t.reshape(TB // 128, 128)
    ie_ref[...] = idx_e.reshape(TB // 128, 128)
    i0_ref[...] = idx0.reshape(TB // 128, 128)
    i1_ref[...] = idx1.reshape(TB // 128, 128)

    part2d = (l_t + l_e + l_0 + l_1).reshape(1, 1)

    @pl.when(pl.program_id(0) == 0)
    def _init():
        loss_ref[...] = part2d

    @pl.when(pl.program_id(0) != 0)
    def _acc():
        loss_ref[...] += part2d


def kernel(x, tw, tb, ew, eb, dw, db, ow, ob, cb_t, cb_e, cb_d0, cb_d1):
    xf = x.reshape(N, D)
    whole = lambda shape: pl.BlockSpec(shape, lambda i: (0, 0))
    row_block = pl.BlockSpec((TB, D), lambda i: (i, 0))
    idx_block = pl.BlockSpec((TB // 128, 128), lambda i: (i, 0))

    out, it, ie, i0, i1, losssum = pl.pallas_call(
        _body,
        grid=(GRID,),
        in_specs=[
            row_block,
            whole((D, D)), whole((1, D)),
            whole((D, D)), whole((1, D)),
            whole((D, D)), whole((1, D)),
            whole((3 * D, D)), whole((1, D)),
            whole((KT, D)), whole((KE, D)),
            whole((KD, D)), whole((KD, D)),
        ],
        out_specs=[
            row_block,
            idx_block, idx_block, idx_block, idx_block,
            pl.BlockSpec((1, 1), lambda i: (0, 0)),
        ],
        out_shape=[
            jax.ShapeDtypeStruct((N, D), _F32),
            jax.ShapeDtypeStruct((N // 128, 128), jnp.int32),
            jax.ShapeDtypeStruct((N // 128, 128), jnp.int32),
            jax.ShapeDtypeStruct((N // 128, 128), jnp.int32),
            jax.ShapeDtypeStruct((N // 128, 128), jnp.int32),
            jax.ShapeDtypeStruct((1, 1), _F32),
        ],
        scratch_shapes=[
            pltpu.VMEM((KC, D), _F32),
            pltpu.VMEM((1, KC), _F32),
        ],
        compiler_params=pltpu.CompilerParams(
            dimension_semantics=("arbitrary",)),
    )(xf, tw, tb.reshape(1, D), ew, eb.reshape(1, D),
      dw, db.reshape(1, D), ow, ob.reshape(1, D),
      cb_t, cb_e, cb_d0, cb_d1)

    loss = (1.0 + COMMIT) * losssum[0, 0] / jnp.float32(N * D)
    return (out.reshape(B, T, D),
            it.reshape(B, T), ie.reshape(B, T),
            i0.reshape(B, T), i1.reshape(B, T),
            loss)


# speedup vs baseline: 1.2271x; 1.2271x over previous
import jax
import jax.numpy as jnp
from jax.experimental import pallas as pl
from jax.experimental.pallas import tpu as pltpu

B, T, D = 8, 2048, 512
N = B * T
KT, KE, KD = 1024, 512, 1024
KC = KT + KE + KD + KD
TB = 1024
GRID = N // TB
COMMIT = 0.25

_F32 = jnp.float32


def _pre_body(ow_ref, cbt_ref, cbe_ref, cbd0_ref, cbd1_ref,
              m_ref, sq_ref, cb2_ref, hi_ref, lo_ref):
    ow = ow_ref[...]
    cbt = cbt_ref[...]
    cbe = cbe_ref[...]
    cbd0 = cbd0_ref[...]
    cbd1 = cbd1_ref[...]
    m_ref[0:KT] = jnp.dot(cbt, ow[0:D], preferred_element_type=_F32)
    m_ref[KT:KT + KE] = jnp.dot(cbe, ow[D:2 * D], preferred_element_type=_F32)
    m_ref[KT + KE:KT + KE + KD] = jnp.dot(cbd0, ow[2 * D:3 * D],
                                          preferred_element_type=_F32)
    m_ref[KT + KE + KD:KC] = jnp.dot(cbd1, ow[2 * D:3 * D],
                                     preferred_element_type=_F32)
    sq_ref[0:1, 0:KT] = jnp.sum(cbt * cbt, axis=1).reshape(1, KT)
    sq_ref[0:1, KT:KT + KE] = jnp.sum(cbe * cbe, axis=1).reshape(1, KE)
    sq_ref[0:1, KT + KE:KT + KE + KD] = (
        jnp.sum(cbd0 * cbd0, axis=1).reshape(1, KD))
    sq_ref[0:1, KT + KE + KD:KC] = (
        jnp.sum(cbd1 * cbd1, axis=1).reshape(1, KD))
    cb2_ref[0:KT] = 2.0 * cbt
    cb2_ref[KT:KT + KE] = 2.0 * cbe
    cb2_ref[KT + KE:KT + KE + KD] = 2.0 * cbd0
    cb2_ref[KT + KE + KD:KC] = 2.0 * cbd1
    hi = cbd0.astype(jnp.bfloat16).astype(_F32)
    hi_ref[...] = hi
    lo_ref[...] = cbd0 - hi


def _vq_argmin(f, cb2, cbsq):
    rowsq = jnp.sum(f * f, axis=1, keepdims=True)
    s2 = jax.lax.dot_general(f, cb2, (((1,), (1,)), ((), ())),
                             preferred_element_type=_F32)
    d = rowsq + cbsq - s2
    dmin = jnp.min(d, axis=1, keepdims=True)
    iota = jax.lax.broadcasted_iota(jnp.int32, d.shape, 1)
    idx = jnp.min(jnp.where(d == dmin, iota, jnp.int32(d.shape[1])), axis=1)
    oh = (iota == idx[:, None]).astype(_F32)
    return idx, oh, jnp.sum(dmin)


def _body(x_ref, w_ref, b_ref, ob_ref,
          m_ref, sq_ref, cb2_ref, hi_ref, lo_ref,
          out_ref, it_ref, ie_ref, i0_ref, i1_ref, loss_ref):
    xb = x_ref[...]
    feats = jnp.dot(xb, w_ref[...], preferred_element_type=_F32) + b_ref[...]
    tf = feats[:, 0:D]
    ef = feats[:, D:2 * D]
    df = feats[:, 2 * D:3 * D]

    idx_t, oh_t, l_t = _vq_argmin(tf, cb2_ref[0:KT], sq_ref[0:1, 0:KT])
    idx_e, oh_e, l_e = _vq_argmin(ef, cb2_ref[KT:KT + KE],
                                  sq_ref[0:1, KT:KT + KE])

    idx0, oh0, l_0 = _vq_argmin(df, cb2_ref[KT + KE:KT + KE + KD],
                                sq_ref[0:1, KT + KE:KT + KE + KD])
    dg = lambda a, b: jax.lax.dot_general(
        a, b, (((1,), (0,)), ((), ())), preferred_element_type=_F32)
    q0 = dg(oh0, hi_ref[...]) + dg(oh0, lo_ref[...])
    q0st = df + (q0 - df)
    r = df - q0st
    idx1, oh1, l_1 = _vq_argmin(r, cb2_ref[KT + KE + KD:KC],
                                sq_ref[0:1, KT + KE + KD:KC])

    out = (dg(oh_t, m_ref[0:KT]) + dg(oh_e, m_ref[KT:KT + KE])
           + dg(oh0, m_ref[KT + KE:KT + KE + KD])
           + dg(oh1, m_ref[KT + KE + KD:KC]) + ob_ref[...])
    out_ref[...] = out

    it_ref[...] = idx_t.reshape(TB // 128, 128)
    ie_ref[...] = idx_e.reshape(TB // 128, 128)
    i0_ref[...] = idx0.reshape(TB // 128, 128)
    i1_ref[...] = idx1.reshape(TB // 128, 128)

    part2d = (l_t + l_e + l_0 + l_1).reshape(1, 1)

    @pl.when(pl.program_id(0) == 0)
    def _init():
        loss_ref[...] = part2d

    @pl.when(pl.program_id(0) != 0)
    def _acc():
        loss_ref[...] += part2d


def kernel(x, tw, tb, ew, eb, dw, db, ow, ob, cb_t, cb_e, cb_d0, cb_d1):
    xf = x.reshape(N, D)
    wcat = jnp.concatenate([tw, ew, dw], axis=1)
    bcat = jnp.concatenate([tb, eb, db]).reshape(1, 3 * D)

    m_cat, sq_cat, cb2_cat, d0hi, d0lo = pl.pallas_call(
        _pre_body,
        out_shape=[
            jax.ShapeDtypeStruct((KC, D), _F32),
            jax.ShapeDtypeStruct((1, KC), _F32),
            jax.ShapeDtypeStruct((KC, D), _F32),
            jax.ShapeDtypeStruct((KD, D), _F32),
            jax.ShapeDtypeStruct((KD, D), _F32),
        ],
    )(ow, cb_t, cb_e, cb_d0, cb_d1)

    whole = lambda shape: pl.BlockSpec(shape, lambda i: (0, 0))
    row_block = pl.BlockSpec((TB, D), lambda i: (i, 0))
    idx_block = pl.BlockSpec((TB // 128, 128), lambda i: (i, 0))

    out, it, ie, i0, i1, losssum = pl.pallas_call(
        _body,
        grid=(GRID,),
        in_specs=[
            row_block,
            whole((D, 3 * D)), whole((1, 3 * D)),
            whole((1, D)),
            whole((KC, D)), whole((1, KC)),
            whole((KC, D)),
            whole((KD, D)), whole((KD, D)),
        ],
        out_specs=[
            row_block,
            idx_block, idx_block, idx_block, idx_block,
            pl.BlockSpec((1, 1), lambda i: (0, 0)),
        ],
        out_shape=[
            jax.ShapeDtypeStruct((N, D), _F32),
            jax.ShapeDtypeStruct((N // 128, 128), jnp.int32),
            jax.ShapeDtypeStruct((N // 128, 128), jnp.int32),
            jax.ShapeDtypeStruct((N // 128, 128), jnp.int32),
            jax.ShapeDtypeStruct((N // 128, 128), jnp.int32),
            jax.ShapeDtypeStruct((1, 1), _F32),
        ],
        compiler_params=pltpu.CompilerParams(
            dimension_semantics=("arbitrary",)),
    )(xf, wcat, bcat, ob.reshape(1, D),
      m_cat, sq_cat, cb2_cat, d0hi, d0lo)

    loss = (1.0 + COMMIT) * losssum[0, 0] / jnp.float32(N * D)
    return (out.reshape(B, T, D),
            it.reshape(B, T), ie.reshape(B, T),
            i0.reshape(B, T), i1.reshape(B, T),
            loss)


# speedup vs baseline: 1.2796x; 1.0428x over previous
import jax
import jax.numpy as jnp
from jax.experimental import pallas as pl
from jax.experimental.pallas import tpu as pltpu

B, T, D = 8, 2048, 512
N = B * T
KT, KE, KD = 1024, 512, 1024
KC = KT + KE + KD + KD
KM = KT + KE + KD
TB = 1024
GRID = N // TB
COMMIT = 0.25

_F32 = jnp.float32


def _pre_body(tw_ref, ew_ref, dw_ref, tb_ref, eb_ref, db_ref,
              ow_ref, cbt_ref, cbe_ref, cbd0_ref, cbd1_ref,
              w_ref, b_ref, m_ref, sq_ref, cb2_ref, hi_ref, lo_ref):
    w_ref[:, 0:D] = tw_ref[...]
    w_ref[:, D:2 * D] = ew_ref[...]
    w_ref[:, 2 * D:3 * D] = dw_ref[...]
    b_ref[0:1, 0:D] = tb_ref[...]
    b_ref[0:1, D:2 * D] = eb_ref[...]
    b_ref[0:1, 2 * D:3 * D] = db_ref[...]
    ow = ow_ref[...]
    cbt = cbt_ref[...]
    cbe = cbe_ref[...]
    cbd0 = cbd0_ref[...]
    cbd1 = cbd1_ref[...]
    m_ref[0:KT] = jnp.dot(cbt, ow[0:D],
                          preferred_element_type=_F32).astype(jnp.bfloat16)
    m_ref[KT:KT + KE] = jnp.dot(cbe, ow[D:2 * D],
                                preferred_element_type=_F32).astype(jnp.bfloat16)
    m_ref[KT + KE:KM] = jnp.dot(cbd1, ow[2 * D:3 * D],
                                preferred_element_type=_F32).astype(jnp.bfloat16)
    sq_ref[0:1, 0:KT] = jnp.sum(cbt * cbt, axis=1).reshape(1, KT)
    sq_ref[0:1, KT:KT + KE] = jnp.sum(cbe * cbe, axis=1).reshape(1, KE)
    sq_ref[0:1, KT + KE:KT + KE + KD] = (
        jnp.sum(cbd0 * cbd0, axis=1).reshape(1, KD))
    sq_ref[0:1, KT + KE + KD:KC] = (
        jnp.sum(cbd1 * cbd1, axis=1).reshape(1, KD))
    cb2_ref[0:KT] = 2.0 * cbt
    cb2_ref[KT:KT + KE] = 2.0 * cbe
    cb2_ref[KT + KE:KT + KE + KD] = 2.0 * cbd0
    cb2_ref[KT + KE + KD:KC] = 2.0 * cbd1
    hi = cbd0.astype(jnp.bfloat16).astype(_F32)
    hi_ref[...] = hi
    lo_ref[...] = cbd0 - hi


def _vq_argmin(f, cb2, cbsq, oh_dtype=jnp.bfloat16):
    rowsq = jnp.sum(f * f, axis=1, keepdims=True)
    s2 = jax.lax.dot_general(f, cb2, (((1,), (1,)), ((), ())),
                             preferred_element_type=_F32)
    d = rowsq + cbsq - s2
    dmin = jnp.min(d, axis=1, keepdims=True)
    iota = jax.lax.broadcasted_iota(jnp.int32, d.shape, 1)
    idx = jnp.min(jnp.where(d == dmin, iota, jnp.int32(d.shape[1])), axis=1)
    oh = (iota == idx[:, None]).astype(oh_dtype)
    return idx, oh, jnp.sum(dmin)


def _body(x_ref, w_ref, b_ref, ow2_ref, ob_ref,
          m_ref, sq_ref, cb2_ref, hi_ref, lo_ref,
          out_ref, it_ref, ie_ref, i0_ref, i1_ref, loss_ref):
    xb = x_ref[...]
    feats = jnp.dot(xb, w_ref[...], preferred_element_type=_F32) + b_ref[...]
    tf = feats[:, 0:D]
    ef = feats[:, D:2 * D]
    df = feats[:, 2 * D:3 * D]

    idx_t, oh_t, l_t = _vq_argmin(tf, cb2_ref[0:KT], sq_ref[0:1, 0:KT])
    idx_e, oh_e, l_e = _vq_argmin(ef, cb2_ref[KT:KT + KE],
                                  sq_ref[0:1, KT:KT + KE])

    idx0, oh0, l_0 = _vq_argmin(df, cb2_ref[KT + KE:KT + KE + KD],
                                sq_ref[0:1, KT + KE:KT + KE + KD],
                                oh_dtype=_F32)
    dg = lambda a, b: jax.lax.dot_general(
        a, b, (((1,), (0,)), ((), ())), preferred_element_type=_F32)
    q0 = dg(oh0, hi_ref[...]) + dg(oh0, lo_ref[...])
    q0st = df + (q0 - df)
    r = df - q0st
    idx1, oh1, l_1 = _vq_argmin(r, cb2_ref[KT + KE + KD:KC],
                                sq_ref[0:1, KT + KE + KD:KC])

    out = (dg(oh_t, m_ref[0:KT]) + dg(oh_e, m_ref[KT:KT + KE])
           + dg(oh1, m_ref[KT + KE:KM])
           + jnp.dot(q0st, ow2_ref[...], preferred_element_type=_F32)
           + ob_ref[...])
    out_ref[...] = out

    it_ref[...] = idx_t.reshape(TB // 128, 128)
    ie_ref[...] = idx_e.reshape(TB // 128, 128)
    i0_ref[...] = idx0.reshape(TB // 128, 128)
    i1_ref[...] = idx1.reshape(TB // 128, 128)

    part2d = (l_t + l_e + l_0 + l_1).reshape(1, 1)

    @pl.when(pl.program_id(0) == 0)
    def _init():
        loss_ref[...] = part2d

    @pl.when(pl.program_id(0) != 0)
    def _acc():
        loss_ref[...] += part2d


def kernel(x, tw, tb, ew, eb, dw, db, ow, ob, cb_t, cb_e, cb_d0, cb_d1):
    xf = x.reshape(N, D)

    wcat, bcat, m_cat, sq_cat, cb2_cat, d0hi, d0lo = pl.pallas_call(
        _pre_body,
        out_shape=[
            jax.ShapeDtypeStruct((D, 3 * D), _F32),
            jax.ShapeDtypeStruct((1, 3 * D), _F32),
            jax.ShapeDtypeStruct((KM, D), jnp.bfloat16),
            jax.ShapeDtypeStruct((1, KC), _F32),
            jax.ShapeDtypeStruct((KC, D), _F32),
            jax.ShapeDtypeStruct((KD, D), _F32),
            jax.ShapeDtypeStruct((KD, D), _F32),
        ],
    )(tw, ew, dw, tb.reshape(1, D), eb.reshape(1, D), db.reshape(1, D),
      ow, cb_t, cb_e, cb_d0, cb_d1)

    whole = lambda shape: pl.BlockSpec(shape, lambda i: (0, 0))
    row_block = pl.BlockSpec((TB, D), lambda i: (i, 0))
    idx_block = pl.BlockSpec((TB // 128, 128), lambda i: (i, 0))

    out, it, ie, i0, i1, losssum = pl.pallas_call(
        _body,
        grid=(GRID,),
        in_specs=[
            row_block,
            whole((D, 3 * D)), whole((1, 3 * D)),
            whole((D, D)), whole((1, D)),
            whole((KM, D)), whole((1, KC)),
            whole((KC, D)),
            whole((KD, D)), whole((KD, D)),
        ],
        out_specs=[
            row_block,
            idx_block, idx_block, idx_block, idx_block,
            pl.BlockSpec((1, 1), lambda i: (0, 0)),
        ],
        out_shape=[
            jax.ShapeDtypeStruct((N, D), _F32),
            jax.ShapeDtypeStruct((N // 128, 128), jnp.int32),
            jax.ShapeDtypeStruct((N // 128, 128), jnp.int32),
            jax.ShapeDtypeStruct((N // 128, 128), jnp.int32),
            jax.ShapeDtypeStruct((N // 128, 128), jnp.int32),
            jax.ShapeDtypeStruct((1, 1), _F32),
        ],
        compiler_params=pltpu.CompilerParams(
            dimension_semantics=("arbitrary",)),
    )(xf, wcat, bcat, ow[2 * D:3 * D], ob.reshape(1, D),
      m_cat, sq_cat, cb2_cat, d0hi, d0lo)

    loss = (1.0 + COMMIT) * losssum[0, 0] / jnp.float32(N * D)
    return (out.reshape(B, T, D),
            it.reshape(B, T), ie.reshape(B, T),
            i0.reshape(B, T), i1.reshape(B, T),
            loss)


# speedup vs baseline: 1.2836x; 1.0031x over previous
import jax
import jax.numpy as jnp
from jax.experimental import pallas as pl
from jax.experimental.pallas import tpu as pltpu

B, T, D = 8, 2048, 512
N = B * T
KT, KE, KD = 1024, 512, 1024
KC = KT + KE + KD + KD
KM = KT + KE + KD
TB = 1024
GRID = N // TB
COMMIT = 0.25

_F32 = jnp.float32


def _pre_body(tw_ref, ew_ref, dw_ref, tb_ref, eb_ref, db_ref,
              ow_ref, cbt_ref, cbe_ref, cbd0_ref, cbd1_ref,
              w_ref, b_ref, m_ref, sq_ref, cb2_ref, hi_ref, lo_ref):
    w_ref[:, 0:D] = tw_ref[...]
    w_ref[:, D:2 * D] = ew_ref[...]
    w_ref[:, 2 * D:3 * D] = dw_ref[...]
    b_ref[0:1, 0:D] = tb_ref[...]
    b_ref[0:1, D:2 * D] = eb_ref[...]
    b_ref[0:1, 2 * D:3 * D] = db_ref[...]
    ow = ow_ref[...]
    cbt = cbt_ref[...]
    cbe = cbe_ref[...]
    cbd0 = cbd0_ref[...]
    cbd1 = cbd1_ref[...]
    m_ref[0:KT] = jnp.dot(cbt, ow[0:D],
                          preferred_element_type=_F32).astype(jnp.bfloat16)
    m_ref[KT:KT + KE] = jnp.dot(cbe, ow[D:2 * D],
                                preferred_element_type=_F32).astype(jnp.bfloat16)
    m_ref[KT + KE:KM] = jnp.dot(cbd1, ow[2 * D:3 * D],
                                preferred_element_type=_F32).astype(jnp.bfloat16)
    sq_ref[0:1, 0:KT] = jnp.sum(cbt * cbt, axis=1).reshape(1, KT)
    sq_ref[0:1, KT:KT + KE] = jnp.sum(cbe * cbe, axis=1).reshape(1, KE)
    sq_ref[0:1, KT + KE:KT + KE + KD] = (
        jnp.sum(cbd0 * cbd0, axis=1).reshape(1, KD))
    sq_ref[0:1, KT + KE + KD:KC] = (
        jnp.sum(cbd1 * cbd1, axis=1).reshape(1, KD))
    cb2_ref[0:KT] = 2.0 * cbt
    cb2_ref[KT:KT + KE] = 2.0 * cbe
    cb2_ref[KT + KE:KT + KE + KD] = 2.0 * cbd0
    cb2_ref[KT + KE + KD:KC] = 2.0 * cbd1
    hi = cbd0.astype(jnp.bfloat16).astype(_F32)
    hi_ref[...] = hi
    lo_ref[...] = cbd0 - hi


def _vq_argmin(f, cb2, cbsq, oh_dtype=jnp.bfloat16):
    rowsq = jnp.sum(f * f, axis=1, keepdims=True)
    s2 = jax.lax.dot_general(f, cb2, (((1,), (1,)), ((), ())),
                             preferred_element_type=_F32)
    d = rowsq + cbsq - s2
    dmin = jnp.min(d, axis=1, keepdims=True)
    iota = jax.lax.broadcasted_iota(jnp.int32, d.shape, 1)
    idx = jnp.min(jnp.where(d == dmin, iota, jnp.int32(d.shape[1])), axis=1)
    oh = (iota == idx[:, None]).astype(oh_dtype)
    return idx, oh, jnp.sum(dmin)


def _body(x_ref, w_ref, b_ref, ow2_ref, ob_ref,
          m_ref, sq_ref, cb2_ref, hi_ref, lo_ref,
          out_ref, it_ref, ie_ref, i0_ref, i1_ref, loss_ref):
    xb = x_ref[...]
    feats = jnp.dot(xb, w_ref[...], preferred_element_type=_F32) + b_ref[...]
    tf = feats[:, 0:D]
    ef = feats[:, D:2 * D]
    df = feats[:, 2 * D:3 * D]

    idx_t, oh_t, l_t = _vq_argmin(tf, cb2_ref[0:KT], sq_ref[0:1, 0:KT])
    idx_e, oh_e, l_e = _vq_argmin(ef, cb2_ref[KT:KT + KE],
                                  sq_ref[0:1, KT:KT + KE])

    idx0, oh0, l_0 = _vq_argmin(df, cb2_ref[KT + KE:KT + KE + KD],
                                sq_ref[0:1, KT + KE:KT + KE + KD],
                                oh_dtype=_F32)
    dg = lambda a, b: jax.lax.dot_general(
        a, b, (((1,), (0,)), ((), ())), preferred_element_type=_F32)
    q0 = dg(oh0, hi_ref[...]) + dg(oh0, lo_ref[...])
    q0st = df + (q0 - df)
    r = df - q0st
    idx1, oh1, l_1 = _vq_argmin(r, cb2_ref[KT + KE + KD:KC],
                                sq_ref[0:1, KT + KE + KD:KC])

    out = (dg(oh_t, m_ref[0:KT]) + dg(oh_e, m_ref[KT:KT + KE])
           + dg(oh1, m_ref[KT + KE:KM])
           + jnp.dot(q0st, ow2_ref[...], preferred_element_type=_F32)
           + ob_ref[...])
    out_ref[...] = out

    it_ref[...] = idx_t.reshape(TB // 128, 128)
    ie_ref[...] = idx_e.reshape(TB // 128, 128)
    i0_ref[...] = idx0.reshape(TB // 128, 128)
    i1_ref[...] = idx1.reshape(TB // 128, 128)

    part = l_t + l_e + l_0 + l_1
    loss_ref[...] = jnp.broadcast_to(part.reshape(1, 1, 1), (1, 1, 128))


def kernel(x, tw, tb, ew, eb, dw, db, ow, ob, cb_t, cb_e, cb_d0, cb_d1):
    xf = x.reshape(N, D)

    wcat, bcat, m_cat, sq_cat, cb2_cat, d0hi, d0lo = pl.pallas_call(
        _pre_body,
        out_shape=[
            jax.ShapeDtypeStruct((D, 3 * D), _F32),
            jax.ShapeDtypeStruct((1, 3 * D), _F32),
            jax.ShapeDtypeStruct((KM, D), jnp.bfloat16),
            jax.ShapeDtypeStruct((1, KC), _F32),
            jax.ShapeDtypeStruct((KC, D), _F32),
            jax.ShapeDtypeStruct((KD, D), _F32),
            jax.ShapeDtypeStruct((KD, D), _F32),
        ],
    )(tw, ew, dw, tb.reshape(1, D), eb.reshape(1, D), db.reshape(1, D),
      ow, cb_t, cb_e, cb_d0, cb_d1)

    whole = lambda shape: pl.BlockSpec(shape, lambda i: (0, 0))
    row_block = pl.BlockSpec((TB, D), lambda i: (i, 0))
    idx_block = pl.BlockSpec((TB // 128, 128), lambda i: (i, 0))

    out, it, ie, i0, i1, losssum = pl.pallas_call(
        _body,
        grid=(GRID,),
        in_specs=[
            row_block,
            whole((D, 3 * D)), whole((1, 3 * D)),
            whole((D, D)), whole((1, D)),
            whole((KM, D)), whole((1, KC)),
            whole((KC, D)),
            whole((KD, D)), whole((KD, D)),
        ],
        out_specs=[
            row_block,
            idx_block, idx_block, idx_block, idx_block,
            pl.BlockSpec((1, 1, 128), lambda i: (i, 0, 0)),
        ],
        out_shape=[
            jax.ShapeDtypeStruct((N, D), _F32),
            jax.ShapeDtypeStruct((N // 128, 128), jnp.int32),
            jax.ShapeDtypeStruct((N // 128, 128), jnp.int32),
            jax.ShapeDtypeStruct((N // 128, 128), jnp.int32),
            jax.ShapeDtypeStruct((N // 128, 128), jnp.int32),
            jax.ShapeDtypeStruct((GRID, 1, 128), _F32),
        ],
        compiler_params=pltpu.CompilerParams(
            dimension_semantics=("parallel",)),
    )(xf, wcat, bcat, ow[2 * D:3 * D], ob.reshape(1, D),
      m_cat, sq_cat, cb2_cat, d0hi, d0lo)

    loss = (1.0 + COMMIT) * jnp.sum(losssum[:, 0, 0]) / jnp.float32(N * D)
    return (out.reshape(B, T, D),
            it.reshape(B, T), ie.reshape(B, T),
            i0.reshape(B, T), i1.reshape(B, T),
            loss)
